# baseline (speedup 1.0000x reference)
import functools

import jax
import jax.numpy as jnp
from jax.experimental import pallas as pl

_B = 4
_S = 8192
_D = 64
_F = 128
_E = 4
_EF = _E * _F
_TB = 1024


def _moe_body(x_ref, wall_ref, w2_ref, o_ref):
    xb = x_ref[...]
    hall = jnp.dot(xb, wall_ref[...], preferred_element_type=jnp.float32)
    h1 = hall[:, :_EF]
    h3 = hall[:, _EF:2 * _EF]
    lg = hall[:, 2 * _EF:]

    lane = jax.lax.broadcasted_iota(jnp.int32, lg.shape, 1)
    neg_inf = jnp.float32(-jnp.inf)
    lgm = jnp.where(lane < _E, lg, neg_inf)
    m1 = jnp.max(lgm, axis=1, keepdims=True)
    i1 = jnp.min(jnp.where(lgm == m1, lane, _F), axis=1, keepdims=True)
    lg2 = jnp.where(lane == i1, neg_inf, lgm)
    m2 = jnp.max(lg2, axis=1, keepdims=True)
    i2 = jnp.min(jnp.where(lg2 == m2, lane, _F), axis=1, keepdims=True)
    e2 = jnp.exp(m2 - m1)
    s = 1.0 + e2
    g1 = 1.0 / s
    g2 = e2 / s

    h = h1 * jax.nn.sigmoid(h1) * h3
    expert = jax.lax.broadcasted_iota(jnp.int32, h.shape, 1) // _F
    gate = jnp.where(expert == i1, g1, 0.0) + jnp.where(expert == i2, g2, 0.0)
    o_ref[...] = jnp.dot(h * gate, w2_ref[...],
                         preferred_element_type=jnp.float32)


@functools.partial(jax.jit, static_argnames=())
def kernel(x, Wr, W1, W2, W3):
    b, s, d = x.shape
    t = b * s
    xt = x.reshape(t, d)
    w1c = W1.transpose(1, 0, 2).reshape(d, _EF)
    w3c = W3.transpose(1, 0, 2).reshape(d, _EF)
    wr_pad = jnp.pad(Wr, ((0, 0), (0, _F - _E)))
    wall = jnp.concatenate([w1c, w3c, wr_pad], axis=1)
    w2c = W2.reshape(_EF, d)

    out = pl.pallas_call(
        _moe_body,
        grid=(t // _TB,),
        in_specs=[
            pl.BlockSpec((_TB, d), lambda i: (i, 0)),
            pl.BlockSpec((d, 2 * _EF + _F), lambda i: (0, 0)),
            pl.BlockSpec((_EF, d), lambda i: (0, 0)),
        ],
        out_specs=pl.BlockSpec((_TB, d), lambda i: (i, 0)),
        out_shape=jax.ShapeDtypeStruct((t, d), jnp.float32),
    )(xt, wall, w2c)
    return out.reshape(b, s, d)


# speedup vs baseline: 1.1712x; 1.1712x over previous
import jax
import jax.numpy as jnp
from jax.experimental import pallas as pl

_D = 64
_F = 128
_E = 4
_TB = 1024


def _moe_body(x_ref, wrp_ref, w1_ref, w3_ref, w2_ref, o_ref):
    xb = x_ref[...]
    lg = jnp.dot(xb, wrp_ref[...], preferred_element_type=jnp.float32)

    lane = jax.lax.broadcasted_iota(jnp.int32, lg.shape, 1)
    neg_inf = jnp.float32(-jnp.inf)
    lgm = jnp.where(lane < _E, lg, neg_inf)
    m1 = jnp.max(lgm, axis=1, keepdims=True)
    i1 = jnp.min(jnp.where(lgm == m1, lane, _F), axis=1, keepdims=True)
    lg2 = jnp.where(lane == i1, neg_inf, lgm)
    m2 = jnp.max(lg2, axis=1, keepdims=True)
    i2 = jnp.min(jnp.where(lg2 == m2, lane, _F), axis=1, keepdims=True)
    e2 = jnp.exp(m2 - m1)
    s = 1.0 + e2
    g1 = 1.0 / s
    g2 = e2 / s

    acc = jnp.zeros((xb.shape[0], _D), jnp.float32)
    for e in range(_E):
        h1 = jnp.dot(xb, w1_ref[e], preferred_element_type=jnp.float32)
        h3 = jnp.dot(xb, w3_ref[e], preferred_element_type=jnp.float32)
        h = h1 * jax.nn.sigmoid(h1) * h3
        y = jnp.dot(h, w2_ref[e], preferred_element_type=jnp.float32)
        ge = jnp.where(i1 == e, g1, 0.0) + jnp.where(i2 == e, g2, 0.0)
        acc = acc + y * ge
    o_ref[...] = acc


def kernel(x, Wr, W1, W2, W3):
    b, s, d = x.shape
    t = b * s
    xt = x.reshape(t, d)
    wrp = jnp.pad(Wr, ((0, 0), (0, _F - _E)))

    out = pl.pallas_call(
        _moe_body,
        grid=(t // _TB,),
        in_specs=[
            pl.BlockSpec((_TB, d), lambda i: (i, 0)),
            pl.BlockSpec((d, _F), lambda i: (0, 0)),
            pl.BlockSpec((_E, _D, _F), lambda i: (0, 0, 0)),
            pl.BlockSpec((_E, _D, _F), lambda i: (0, 0, 0)),
            pl.BlockSpec((_E, _F, _D), lambda i: (0, 0, 0)),
        ],
        out_specs=pl.BlockSpec((_TB, d), lambda i: (i, 0)),
        out_shape=jax.ShapeDtypeStruct((t, d), jnp.float32),
    )(xt, wrp, W1, W3, W2)
    return out.reshape(b, s, d)


# speedup vs baseline: 1.2196x; 1.0413x over previous
import jax
import jax.numpy as jnp
from jax.experimental import pallas as pl

_D = 64
_F = 128
_E = 4
_TB = 4096


def _moe_body(x_ref, wrp_ref, w1_ref, w3_ref, w2_ref, o_ref):
    xb = x_ref[...]
    lg = jnp.dot(xb, wrp_ref[...], preferred_element_type=jnp.float32)

    lane = jax.lax.broadcasted_iota(jnp.int32, lg.shape, 1)
    neg_inf = jnp.float32(-jnp.inf)
    lgm = jnp.where(lane < _E, lg, neg_inf)
    m1 = jnp.max(lgm, axis=1, keepdims=True)
    i1 = jnp.min(jnp.where(lgm == m1, lane, _F), axis=1, keepdims=True)
    lg2 = jnp.where(lane == i1, neg_inf, lgm)
    m2 = jnp.max(lg2, axis=1, keepdims=True)
    i2 = jnp.min(jnp.where(lg2 == m2, lane, _F), axis=1, keepdims=True)
    e2 = jnp.exp(m2 - m1)
    s = 1.0 + e2
    g1 = 1.0 / s
    g2 = e2 / s

    acc = jnp.zeros((xb.shape[0], _D), jnp.float32)
    for e in range(_E):
        h1 = jnp.dot(xb, w1_ref[e], preferred_element_type=jnp.float32)
        h3 = jnp.dot(xb, w3_ref[e], preferred_element_type=jnp.float32)
        h = h1 * jax.nn.sigmoid(h1) * h3
        y = jnp.dot(h, w2_ref[e], preferred_element_type=jnp.float32)
        ge = jnp.where(i1 == e, g1, 0.0) + jnp.where(i2 == e, g2, 0.0)
        acc = acc + y * ge
    o_ref[...] = acc


def kernel(x, Wr, W1, W2, W3):
    b, s, d = x.shape
    t = b * s
    xt = x.reshape(t, d)
    wrp = jnp.pad(Wr, ((0, 0), (0, _F - _E)))

    out = pl.pallas_call(
        _moe_body,
        grid=(t // _TB,),
        in_specs=[
            pl.BlockSpec((_TB, d), lambda i: (i, 0)),
            pl.BlockSpec((d, _F), lambda i: (0, 0)),
            pl.BlockSpec((_E, _D, _F), lambda i: (0, 0, 0)),
            pl.BlockSpec((_E, _D, _F), lambda i: (0, 0, 0)),
            pl.BlockSpec((_E, _F, _D), lambda i: (0, 0, 0)),
        ],
        out_specs=pl.BlockSpec((_TB, d), lambda i: (i, 0)),
        out_shape=jax.ShapeDtypeStruct((t, d), jnp.float32),
    )(xt, wrp, W1, W3, W2)
    return out.reshape(b, s, d)


# speedup vs baseline: 1.3106x; 1.0746x over previous
import jax
import jax.numpy as jnp
from jax.experimental import pallas as pl

_D = 64
_F = 128
_E = 4
_TB = 4096


def _moe_body(x_ref, wr_ref, w1_ref, w3_ref, w2_ref, o_ref):
    xb = x_ref[0]
    wrp = jnp.pad(wr_ref[...], ((0, 0), (0, _F - _E)))
    lg = jnp.dot(xb, wrp, preferred_element_type=jnp.float32)

    lane = jax.lax.broadcasted_iota(jnp.int32, lg.shape, 1)
    neg_inf = jnp.float32(-jnp.inf)
    lgm = jnp.where(lane < _E, lg, neg_inf)
    m1 = jnp.max(lgm, axis=1, keepdims=True)
    i1 = jnp.min(jnp.where(lgm == m1, lane, _F), axis=1, keepdims=True)
    lg2 = jnp.where(lane == i1, neg_inf, lgm)
    m2 = jnp.max(lg2, axis=1, keepdims=True)
    i2 = jnp.min(jnp.where(lg2 == m2, lane, _F), axis=1, keepdims=True)
    e2 = jnp.exp(m2 - m1)
    s = 1.0 + e2
    g1 = 1.0 / s
    g2 = e2 / s

    acc = jnp.zeros((xb.shape[0], _D), jnp.float32)
    for e in range(_E):
        h1 = jnp.dot(xb, w1_ref[e], preferred_element_type=jnp.float32)
        h3 = jnp.dot(xb, w3_ref[e], preferred_element_type=jnp.float32)
        h = h1 * jax.nn.sigmoid(h1) * h3
        y = jnp.dot(h, w2_ref[e], preferred_element_type=jnp.float32)
        ge = jnp.where(i1 == e, g1, 0.0) + jnp.where(i2 == e, g2, 0.0)
        acc = acc + y * ge
    o_ref[0] = acc


def kernel(x, Wr, W1, W2, W3):
    b, s, d = x.shape
    sb = s // _TB

    out = pl.pallas_call(
        _moe_body,
        grid=(b * sb,),
        in_specs=[
            pl.BlockSpec((1, _TB, d), lambda i: (i // sb, i % sb, 0)),
            pl.BlockSpec((_D, _E), lambda i: (0, 0)),
            pl.BlockSpec((_E, _D, _F), lambda i: (0, 0, 0)),
            pl.BlockSpec((_E, _D, _F), lambda i: (0, 0, 0)),
            pl.BlockSpec((_E, _F, _D), lambda i: (0, 0, 0)),
        ],
        out_specs=pl.BlockSpec((1, _TB, d), lambda i: (i // sb, i % sb, 0)),
        out_shape=jax.ShapeDtypeStruct((b, s, d), jnp.float32),
    )(x, Wr, W1, W3, W2)
    return out


# speedup vs baseline: 2.6468x; 2.0196x over previous
import jax
import jax.numpy as jnp
from jax.experimental import pallas as pl

_D = 64
_F = 128
_E = 4
_STB = 4096


def _moe_body(x_ref, wr_ref, w1_ref, w3_ref, w2_ref, o_ref):
    xb = x_ref[0]
    dn = (((0,), (0,)), ((), ()))
    lg = jax.lax.dot_general(wr_ref[...], xb, dn,
                             preferred_element_type=jnp.float32)

    row = jax.lax.broadcasted_iota(jnp.int32, lg.shape, 0)
    neg_inf = jnp.float32(-jnp.inf)
    m1 = jnp.max(lg, axis=0, keepdims=True)
    i1 = jnp.min(jnp.where(lg == m1, row, _E), axis=0, keepdims=True)
    mask1 = row == i1
    lg2 = jnp.where(mask1, neg_inf, lg)
    m2 = jnp.max(lg2, axis=0, keepdims=True)
    i2 = jnp.min(jnp.where(lg2 == m2, row, _E), axis=0, keepdims=True)
    mask2 = row == i2
    e2 = jnp.exp(m2 - m1)
    g1 = 1.0 / (1.0 + e2)
    g2 = 1.0 - g1
    gt = jnp.where(mask1, g1, 0.0) + jnp.where(mask2, g2, 0.0)

    acc = jnp.zeros((_D, xb.shape[1]), jnp.float32)
    for e in range(_E):
        h1 = jax.lax.dot_general(w1_ref[e], xb, dn,
                                 preferred_element_type=jnp.float32)
        h3 = jax.lax.dot_general(w3_ref[e], xb, dn,
                                 preferred_element_type=jnp.float32)
        h = h1 * jax.nn.sigmoid(h1) * h3
        y = jax.lax.dot_general(w2_ref[e], h, dn,
                                preferred_element_type=jnp.float32)
        acc = acc + y * gt[e:e + 1]
    o_ref[0] = acc


def kernel(x, Wr, W1, W2, W3):
    b, s, d = x.shape
    sb = s // _STB
    xt = jnp.transpose(x, (0, 2, 1))

    out = pl.pallas_call(
        _moe_body,
        grid=(b * sb,),
        in_specs=[
            pl.BlockSpec((1, d, _STB), lambda i: (i // sb, 0, i % sb)),
            pl.BlockSpec((_D, _E), lambda i: (0, 0)),
            pl.BlockSpec((_E, _D, _F), lambda i: (0, 0, 0)),
            pl.BlockSpec((_E, _D, _F), lambda i: (0, 0, 0)),
            pl.BlockSpec((_E, _F, _D), lambda i: (0, 0, 0)),
        ],
        out_specs=pl.BlockSpec((1, d, _STB), lambda i: (i // sb, 0, i % sb)),
        out_shape=jax.ShapeDtypeStruct((b, d, s), jnp.float32),
    )(xt, Wr, W1, W3, W2)
    return jnp.transpose(out, (0, 2, 1))


# speedup vs baseline: 3.1375x; 1.1854x over previous
import jax
import jax.numpy as jnp
from jax.experimental import pallas as pl

_D = 64
_F = 128
_E = 4
_STB = 4096


def _moe_body(x_ref, wr_ref, w1_ref, w3_ref, w2_ref, o_ref):
    xb = x_ref[0]
    dn = (((0,), (0,)), ((), ()))
    lg = jax.lax.dot_general(wr_ref[...], xb, dn,
                             preferred_element_type=jnp.float32)

    row = jax.lax.broadcasted_iota(jnp.int32, lg.shape, 0)
    neg_inf = jnp.float32(-jnp.inf)
    m1 = jnp.max(lg, axis=0, keepdims=True)
    i1 = jnp.min(jnp.where(lg == m1, row, _E), axis=0, keepdims=True)
    mask1 = row == i1
    lg2 = jnp.where(mask1, neg_inf, lg)
    m2 = jnp.max(lg2, axis=0, keepdims=True)
    i2 = jnp.min(jnp.where(lg2 == m2, row, _E), axis=0, keepdims=True)
    mask2 = row == i2
    e2 = jnp.exp(m2 - m1)
    g1 = 1.0 / (1.0 + e2)
    g2 = 1.0 - g1
    gt = jnp.where(mask1, g1, 0.0) + jnp.where(mask2, g2, 0.0)

    acc = jnp.zeros((_D, xb.shape[1]), jnp.float32)
    for e in range(_E):
        h1 = jax.lax.dot_general(w1_ref[e], xb, dn,
                                 preferred_element_type=jnp.float32)
        h3 = jax.lax.dot_general(w3_ref[e], xb, dn,
                                 preferred_element_type=jnp.float32)
        h = h1 * (0.5 * jnp.tanh(0.5 * h1) + 0.5) * h3
        y = jax.lax.dot_general(w2_ref[e], h, dn,
                                preferred_element_type=jnp.float32)
        acc = acc + y * gt[e:e + 1]
    o_ref[0] = acc


def kernel(x, Wr, W1, W2, W3):
    b, s, d = x.shape
    sb = s // _STB
    xt = jnp.transpose(x, (0, 2, 1))

    out = pl.pallas_call(
        _moe_body,
        grid=(b * sb,),
        in_specs=[
            pl.BlockSpec((1, d, _STB), lambda i: (i // sb, 0, i % sb)),
            pl.BlockSpec((_D, _E), lambda i: (0, 0)),
            pl.BlockSpec((_E, _D, _F), lambda i: (0, 0, 0)),
            pl.BlockSpec((_E, _D, _F), lambda i: (0, 0, 0)),
            pl.BlockSpec((_E, _F, _D), lambda i: (0, 0, 0)),
        ],
        out_specs=pl.BlockSpec((1, d, _STB), lambda i: (i // sb, 0, i % sb)),
        out_shape=jax.ShapeDtypeStruct((b, d, s), jnp.float32),
    )(xt, Wr, W1, W3, W2)
    return jnp.transpose(out, (0, 2, 1))
